# Initial kernel scaffold; baseline (speedup 1.0000x reference)
import jax
import jax.numpy as jnp
from jax.experimental import pallas as pl


def kernel(x, edge_index, edge_attr, enc_nW1, enc_nb1, enc_nW2, enc_nb2, enc_eW1, enc_eb1, enc_eW2, enc_eb2, core_eW1, core_eb1, core_eW2, core_eb2, core_nW1, core_nb1, core_nW2, core_nb2, dec_nW1, dec_nb1, dec_nW2, dec_nb2, dec_eW1, dec_eb1, dec_eW2, dec_eb2):
    raise NotImplementedError("write your pallas kernel here")


# speedup vs baseline: 2.8880x; 2.8880x over previous
import functools

import jax
import jax.numpy as jnp
from jax import lax
from jax.experimental import pallas as pl
from jax.experimental.pallas import tpu as pltpu
from jax.experimental.pallas import tpu_sc as plsc

N = 10000
E = 320000
F = 128

NC = 2
NS = 16
NW = NC * NS

EW = E // NW
C = 80
NCHUNK = EW // C
NPAD = 10240
ROWS_PER_TILE = NPAD // NS

_mesh = plsc.VectorSubcoreMesh(core_axis_name="c", subcore_axis_name="s",
                               num_cores=NC, num_subcores=NS)


@functools.partial(
    pl.kernel,
    out_type=(
        jax.ShapeDtypeStruct((E, F), jnp.float32),
        jax.ShapeDtypeStruct((E, F), jnp.float32),
    ),
    mesh=_mesh,
    scratch_types=[
        pltpu.VMEM((NCHUNK, C), jnp.int32),
        pltpu.VMEM((NCHUNK, C), jnp.int32),
        pltpu.VMEM((C, F), jnp.float32),
        pltpu.VMEM((C, F), jnp.float32),
        pltpu.SemaphoreType.DMA,
        pltpu.SemaphoreType.DMA,
    ],
)
def _sc_gather(x_hbm, src2d_hbm, dst2d_hbm, xs_hbm, xd_hbm,
               idx_s, idx_d, rows_s, rows_d, sem_s, sem_d):
    wid = lax.axis_index("s") * NC + lax.axis_index("c")
    pltpu.sync_copy(src2d_hbm.at[wid], idx_s)
    pltpu.sync_copy(dst2d_hbm.at[wid], idx_d)

    def body(j, carry):
        ds = pltpu.async_copy(x_hbm.at[idx_s.at[j]], rows_s, sem_s)
        dd = pltpu.async_copy(x_hbm.at[idx_d.at[j]], rows_d, sem_d)
        ds.wait()
        dd.wait()
        base = wid * EW + j * C
        pltpu.sync_copy(rows_s, xs_hbm.at[pl.ds(base, C)])
        pltpu.sync_copy(rows_d, xd_hbm.at[pl.ds(base, C)])
        return carry

    lax.fori_loop(0, NCHUNK, body, 0)


@functools.partial(
    pl.kernel,
    out_type=jax.ShapeDtypeStruct((NC, NPAD, F), jnp.float32),
    mesh=_mesh,
    scratch_types=[
        pltpu.VMEM((NCHUNK, C), jnp.int32),
        pltpu.VMEM((C, F), jnp.float32),
        pltpu.VMEM_SHARED((NPAD, F), jnp.float32),
    ],
)
def _sc_scatter(e_hbm, dst2d_hbm, zeros_hbm, out_hbm, idx_d, rows, shared):
    cid = lax.axis_index("c")
    sid = lax.axis_index("s")
    wid = sid * NC + cid
    tbase = sid * ROWS_PER_TILE
    pltpu.sync_copy(zeros_hbm.at[pl.ds(tbase, ROWS_PER_TILE)],
                    shared.at[pl.ds(tbase, ROWS_PER_TILE)])
    pltpu.sync_copy(dst2d_hbm.at[wid], idx_d)
    plsc.subcore_barrier()

    def body(j, carry):
        pltpu.sync_copy(e_hbm.at[pl.ds(wid * EW + j * C, C)], rows)
        pltpu.sync_copy(rows, shared.at[idx_d.at[j]], add=True)
        return carry

    lax.fori_loop(0, NCHUNK, body, 0)
    plsc.subcore_barrier()
    pltpu.sync_copy(shared.at[pl.ds(tbase, ROWS_PER_TILE)],
                    out_hbm.at[cid, pl.ds(tbase, ROWS_PER_TILE)])


def _mlp_body(nx, residual_idx, *refs):
    x_refs = refs[:nx]
    w1_refs = refs[nx:2 * nx]
    b1_ref, w2_ref, b2_ref, o_ref = refs[2 * nx:]
    acc = b1_ref[0, :].astype(jnp.float32)
    acc = jnp.zeros_like(o_ref[...]) + acc[None, :]
    for i in range(nx):
        acc = acc + jnp.dot(x_refs[i][...], w1_refs[i][...],
                            preferred_element_type=jnp.float32)
    h = jax.nn.relu(acc)
    out = jnp.dot(h, w2_ref[...], preferred_element_type=jnp.float32)
    out = out + b2_ref[0, :][None, :]
    if residual_idx is not None:
        out = out + x_refs[residual_idx][...]
    o_ref[...] = out


def _mlp(xs, w1s, b1, w2, b2, residual_idx=None, bm=2000):
    nx = len(xs)
    m = xs[0].shape[0]
    h_dim = w2.shape[0]
    o_dim = w2.shape[1]
    grid = (m // bm,)
    in_specs = (
        [pl.BlockSpec((bm, x.shape[1]), lambda i: (i, 0)) for x in xs]
        + [pl.BlockSpec(w.shape, lambda i: (0, 0)) for w in w1s]
        + [pl.BlockSpec((1, h_dim), lambda i: (0, 0)),
           pl.BlockSpec((h_dim, o_dim), lambda i: (0, 0)),
           pl.BlockSpec((1, o_dim), lambda i: (0, 0))]
    )
    return pl.pallas_call(
        functools.partial(_mlp_body, nx, residual_idx),
        grid=grid,
        in_specs=in_specs,
        out_specs=pl.BlockSpec((bm, o_dim), lambda i: (i, 0)),
        out_shape=jax.ShapeDtypeStruct((m, o_dim), jnp.float32),
    )(*xs, *w1s, b1.reshape(1, -1), w2, b2.reshape(1, -1))


def kernel(x, edge_index, edge_attr,
           enc_nW1, enc_nb1, enc_nW2, enc_nb2,
           enc_eW1, enc_eb1, enc_eW2, enc_eb2,
           core_eW1, core_eb1, core_eW2, core_eb2,
           core_nW1, core_nb1, core_nW2, core_nb2,
           dec_nW1, dec_nb1, dec_nW2, dec_nb2,
           dec_eW1, dec_eb1, dec_eW2, dec_eb2):
    x = x.astype(jnp.float32)
    e = edge_attr.astype(jnp.float32)
    src2d = edge_index[0].reshape(NW, NCHUNK, C)
    dst2d = edge_index[1].reshape(NW, NCHUNK, C)
    zeros = jnp.zeros((NPAD, F), jnp.float32)

    x = _mlp([x], [enc_nW1], enc_nb1, enc_nW2, enc_nb2)
    e = _mlp([e], [enc_eW1], enc_eb1, enc_eW2, enc_eb2)

    eW1a = core_eW1[:F]
    eW1b = core_eW1[F:2 * F]
    eW1c = core_eW1[2 * F:]
    nW1a = core_nW1[:F]
    nW1b = core_nW1[F:]

    for _ in range(3):
        xs, xd = _sc_gather(x, src2d, dst2d)
        e = _mlp([xs, xd, e], [eW1a, eW1b, eW1c], core_eb1, core_eW2,
                 core_eb2, residual_idx=2)
        parts = _sc_scatter(e, dst2d, zeros)
        p0 = parts[0, :N]
        p1 = parts[1, :N]
        x = _mlp([x, p0, p1], [nW1a, nW1b, nW1b], core_nb1, core_nW2,
                 core_nb2, residual_idx=0)

    x = _mlp([x], [dec_nW1], dec_nb1, dec_nW2, dec_nb2)
    e = _mlp([e], [dec_eW1], dec_eb1, dec_eW2, dec_eb2)
    return (x, e)
